# baseline (speedup 1.0000x reference)
import functools

import jax
import jax.numpy as jnp
from jax import lax
from jax.experimental import pallas as pl
from jax.experimental.pallas import tpu as pltpu
from jax.experimental.pallas import tpu_sc as plsc

_NEG_INF = float("-inf")


def _make_sc_segmax(n_nodes, n_edges, feat):
  info = plsc.get_sparse_core_info()
  nc, ns = info.num_cores, info.num_subcores
  nw = nc * ns
  nr = nw // 2
  npw = -(-n_nodes // nr)
  n_pad = npw * nr
  half = n_edges // 2
  K = 1600
  assert half % K == 0 and K % 64 == 0
  G = 128
  M = K + G + 16
  assert M % 16 == 0
  vpr = feat // 16

  mesh = plsc.VectorSubcoreMesh(core_axis_name="c", subcore_axis_name="s")

  @functools.partial(
      pl.kernel,
      mesh=mesh,
      compiler_params=pltpu.CompilerParams(needs_layout_passes=False),
      out_type=jax.ShapeDtypeStruct((2 * n_pad * feat,), jnp.float32),
      scratch_types=[
          pltpu.VMEM(((npw + 1) * feat,), jnp.float32),
          pltpu.VMEM((2 * K,), jnp.int32),
          pltpu.VMEM((2 * K,), jnp.int32),
          pltpu.VMEM((M,), jnp.int32),
          pltpu.VMEM((G,), jnp.int32),
          pltpu.VMEM((G,), jnp.int32),
          pltpu.VMEM((G, feat), jnp.float32),
          pltpu.VMEM((G, feat), jnp.float32),
          pltpu.SemaphoreType.DMA,
          pltpu.SemaphoreType.DMA,
          pltpu.SemaphoreType.DMA,
          pltpu.SemaphoreType.DMA,
      ],
  )
  def segmax(xf_hbm, src_hbm, dst_hbm, out_hbm,
             acc_v, srcc_v, dstc_v, mpk_v, gidx_a, gidx_b, drows_a, drows_b,
             sem_s, sem_d, sem_a, sem_b):
    h = lax.axis_index("c")
    rng = lax.axis_index("s")
    lo = rng * npw
    ebase = h * half
    nchunks = half // K

    def init_acc(i, _):
      acc_v[pl.ds(i * 16, 16)] = jnp.full((16,), _NEG_INF, jnp.float32)
      return 0
    lax.fori_loop(0, (npw + 1) * feat // 16, init_acc, 0)

    zeros16 = jnp.zeros((16,), jnp.int32)

    def init_m(i, _):
      mpk_v[pl.ds(i * 16, 16)] = zeros16
      return 0
    lax.fori_loop(0, M // 16, init_m, 0)

    lane_iota = lax.iota(jnp.int32, 16)

    def chunk_copy(t, par, sem, hbm, buf):
      return pltpu.make_async_copy(
          hbm.at[pl.ds(ebase + t * K, K)], buf.at[pl.ds(par * K, K)], sem)

    def start_gather(pos, gidx, drows, sem):
      def unpack(j, _):
        gidx[pl.ds(j * 16, 16)] = mpk_v[pl.ds(pos + j * 16, 16)] >> 14
        return 0
      lax.fori_loop(0, G // 16, unpack, 0)
      pltpu.make_async_copy(xf_hbm.at[gidx], drows, sem).start()

    def accum_group(pos, nvalid, gidx, drows, sem, full):
      pltpu.make_async_copy(xf_hbm.at[gidx], drows, sem).wait()

      def per_16(g, _):
        seg_v = mpk_v[pl.ds(pos + g * 16, 16)] & 0x3FFF
        if full:
          off_v = seg_v - lo
        else:
          m_v = g * 16 + lane_iota
          off_v = jnp.where(m_v < nvalid, seg_v - lo, npw)
        base_v = off_v * feat
        for k in range(vpr):
          for lane in range(16):
            sl = pl.ds(base_v[lane] + k * 16, 16)
            acc_v[sl] = jnp.maximum(
                acc_v[sl], drows[g * 16 + lane, pl.ds(k * 16, 16)])
        return 0
      lax.fori_loop(0, G // 16, per_16, 0)

    chunk_copy(0, 0, sem_s, src_hbm, srcc_v).start()
    chunk_copy(0, 0, sem_d, dst_hbm, dstc_v).start()

    def chunk_body(t, cursor):
      par = lax.rem(t, 2)
      chunk_copy(t, par, sem_s, src_hbm, srcc_v).wait()
      chunk_copy(t, par, sem_d, dst_hbm, dstc_v).wait()

      @pl.when(t + 1 < nchunks)
      def _():
        chunk_copy(t + 1, 1 - par, sem_s, src_hbm, srcc_v).start()
        chunk_copy(t + 1, 1 - par, sem_d, dst_hbm, dstc_v).start()

      cbase = par * K

      def scan_g(g, cur):
        for u in range(4):
          base = cbase + g * 64 + u * 16
          sv = srcc_v[pl.ds(base, 16)]
          dv = dstc_v[pl.ds(base, 16)]
          msk = plsc.bitcast(sv - lo, jnp.uint32) < jnp.uint32(npw)
          packed = sv | (dv << 14)
          mi = jnp.where(msk, jnp.int32(1), jnp.int32(0))
          csum = plsc.cumsum(mi)
          plsc.store_scatter(mpk_v, [cur + csum - mi], packed, mask=msk)
          cur = cur + plsc.all_reduce_population_count(msk)[0]
        return cur
      cursor = lax.fori_loop(0, K // 64, scan_g, cursor)

      nfull = cursor // G

      @pl.when(nfull > 0)
      def _():
        start_gather(0, gidx_a, drows_a, sem_a)

      def flush(m, _):
        gpar = lax.rem(m, 2)

        @pl.when(m + 1 < nfull)
        def _():
          @pl.when(gpar == 0)
          def _():
            start_gather((m + 1) * G, gidx_b, drows_b, sem_b)
          @pl.when(gpar == 1)
          def _():
            start_gather((m + 1) * G, gidx_a, drows_a, sem_a)

        @pl.when(gpar == 0)
        def _():
          accum_group(m * G, G, gidx_a, drows_a, sem_a, True)
        @pl.when(gpar == 1)
        def _():
          accum_group(m * G, G, gidx_b, drows_b, sem_b, True)
        return 0
      lax.fori_loop(0, nfull, flush, 0)

      rem = cursor - nfull * G

      def shift(j, _):
        mpk_v[pl.ds(j * 16, 16)] = mpk_v[pl.ds(nfull * G + j * 16, 16)]
        return 0
      lax.fori_loop(0, jnp.where(nfull > 0, (rem + 15) // 16, 0), shift, 0)
      return rem

    cursor = lax.fori_loop(0, nchunks, chunk_body, jnp.int32(0))

    @pl.when(cursor > 0)
    def _():
      start_gather(0, gidx_a, drows_a, sem_a)
      accum_group(0, cursor, gidx_a, drows_a, sem_a, False)

    pltpu.sync_copy(acc_v.at[pl.ds(0, npw * feat)],
                    out_hbm.at[pl.ds((h * n_pad + lo) * feat, npw * feat)])

  return segmax, n_pad


def _tc_fused(xf, sm_a, sm_b, w_even_t, w_odd_t, b2):
  n, feat = xf.shape
  blk = 2000
  assert n % blk == 0

  def body(xf_ref, sa_ref, sb_ref, we_ref, wo_ref, b_ref, o_ref):
    xb = xf_ref[...]
    sm = jnp.maximum(sa_ref[...], sb_ref[...])
    agg = jnp.where(sm == _NEG_INF, 0.0, sm - xb)
    y = jnp.dot(xb, we_ref[...], preferred_element_type=jnp.float32)
    y = y + jnp.dot(agg, wo_ref[...], preferred_element_type=jnp.float32)
    y = y + b_ref[...]
    o_ref[...] = jnp.maximum(y, 0.0)

  return pl.pallas_call(
      body,
      grid=(n // blk,),
      in_specs=[
          pl.BlockSpec((blk, feat), lambda i: (i, 0)),
          pl.BlockSpec((blk, feat), lambda i: (i, 0)),
          pl.BlockSpec((blk, feat), lambda i: (i, 0)),
          pl.BlockSpec((feat, feat), lambda i: (0, 0)),
          pl.BlockSpec((feat, feat), lambda i: (0, 0)),
          pl.BlockSpec((1, feat), lambda i: (0, 0)),
      ],
      out_specs=pl.BlockSpec((blk, feat), lambda i: (i, 0)),
      out_shape=jax.ShapeDtypeStruct((n, feat), jnp.float32),
  )(xf, sm_a, sm_b, w_even_t, w_odd_t, b2)


def kernel(x, edge_index, W, b):
  bsz, feat, n, _ = x.shape
  n_edges = edge_index.shape[1]
  assert bsz == 1

  xf = jnp.transpose(x[0, :, :, 0])
  src = edge_index[0].astype(jnp.int32)
  dst = edge_index[1].astype(jnp.int32)

  sc_segmax, n_pad = _make_sc_segmax(n, n_edges, feat)
  sm_flat = sc_segmax(xf, src, dst)
  sm2 = sm_flat.reshape(2, n_pad, feat)
  sm_a = sm2[0, :n]
  sm_b = sm2[1, :n]

  w_even_t = jnp.transpose(W[:, 0::2])
  w_odd_t = jnp.transpose(W[:, 1::2])
  y = _tc_fused(xf, sm_a, sm_b, w_even_t, w_odd_t, b.reshape(1, feat))
  return jnp.transpose(y)[None, :, :, None]


# speedup vs baseline: 1.0629x; 1.0629x over previous
import functools

import jax
import jax.numpy as jnp
from jax import lax
from jax.experimental import pallas as pl
from jax.experimental.pallas import tpu as pltpu
from jax.experimental.pallas import tpu_sc as plsc

_NEG_INF = float("-inf")


def _make_sc_segmax(n_nodes, n_edges, feat):
  info = plsc.get_sparse_core_info()
  nc, ns = info.num_cores, info.num_subcores
  nw = nc * ns
  nr = nw // 2
  npw = -(-n_nodes // nr)
  n_pad = npw * nr
  half = n_edges // 2
  K = 1600
  assert half % K == 0 and K % 64 == 0
  G = 128
  M = K + G + 16
  assert M % 16 == 0
  vpr = feat // 16

  mesh = plsc.VectorSubcoreMesh(core_axis_name="c", subcore_axis_name="s")

  @functools.partial(
      pl.kernel,
      mesh=mesh,
      compiler_params=pltpu.CompilerParams(needs_layout_passes=False),
      out_type=jax.ShapeDtypeStruct((2 * n_pad * feat,), jnp.float32),
      scratch_types=[
          pltpu.VMEM(((npw + 1) * feat,), jnp.float32),
          pltpu.VMEM((2 * K,), jnp.int32),
          pltpu.VMEM((2 * K,), jnp.int32),
          pltpu.VMEM((M,), jnp.int32),
          pltpu.VMEM((G,), jnp.int32),
          pltpu.VMEM((G,), jnp.int32),
          pltpu.VMEM((G, feat), jnp.float32),
          pltpu.VMEM((G, feat), jnp.float32),
          pltpu.SemaphoreType.DMA,
          pltpu.SemaphoreType.DMA,
          pltpu.SemaphoreType.DMA,
          pltpu.SemaphoreType.DMA,
      ],
  )
  def segmax(xf_hbm, src_hbm, dst_hbm, out_hbm,
             acc_v, srcc_v, dstc_v, mpk_v, gidx_a, gidx_b, drows_a, drows_b,
             sem_s, sem_d, sem_a, sem_b):
    h = lax.axis_index("c")
    rng = lax.axis_index("s")
    lo = rng * npw
    ebase = h * half
    nchunks = half // K

    def init_acc(i, _):
      acc_v[pl.ds(i * 16, 16)] = jnp.full((16,), _NEG_INF, jnp.float32)
      return 0
    lax.fori_loop(0, (npw + 1) * feat // 16, init_acc, 0)

    zeros16 = jnp.zeros((16,), jnp.int32)

    def init_m(i, _):
      mpk_v[pl.ds(i * 16, 16)] = zeros16
      return 0
    lax.fori_loop(0, M // 16, init_m, 0)

    lane_iota = lax.iota(jnp.int32, 16)

    def chunk_copy(t, par, sem, hbm, buf):
      return pltpu.make_async_copy(
          hbm.at[pl.ds(ebase + t * K, K)], buf.at[pl.ds(par * K, K)], sem)

    def start_gather(pos, gidx, drows, sem):
      def unpack(j, _):
        gidx[pl.ds(j * 16, 16)] = mpk_v[pl.ds(pos + j * 16, 16)] >> 14
        return 0
      lax.fori_loop(0, G // 16, unpack, 0)
      pltpu.make_async_copy(xf_hbm.at[gidx], drows, sem).start()

    def accum_group(pos, nvalid, gidx, drows, sem, full):
      pltpu.make_async_copy(xf_hbm.at[gidx], drows, sem).wait()

      def per_16(g, _):
        seg_v = mpk_v[pl.ds(pos + g * 16, 16)] & 0x3FFF
        if full:
          off_v = seg_v - lo
        else:
          m_v = g * 16 + lane_iota
          off_v = jnp.where(m_v < nvalid, seg_v - lo, npw)
        base_v = off_v * feat
        for lane in range(16):
          base = base_v[lane]
          m = g * 16 + lane
          for k in range(vpr):
            sl = pl.ds(base + k * 16, 16)
            acc_v[sl] = jnp.maximum(acc_v[sl], drows[m, pl.ds(k * 16, 16)])
        return 0
      lax.fori_loop(0, G // 16, per_16, 0)

    chunk_copy(0, 0, sem_s, src_hbm, srcc_v).start()
    chunk_copy(0, 0, sem_d, dst_hbm, dstc_v).start()

    def chunk_body(t, cursor):
      par = lax.rem(t, 2)
      chunk_copy(t, par, sem_s, src_hbm, srcc_v).wait()
      chunk_copy(t, par, sem_d, dst_hbm, dstc_v).wait()

      @pl.when(t + 1 < nchunks)
      def _():
        chunk_copy(t + 1, 1 - par, sem_s, src_hbm, srcc_v).start()
        chunk_copy(t + 1, 1 - par, sem_d, dst_hbm, dstc_v).start()

      cbase = par * K

      def scan_g(g, cur):
        for u in range(4):
          base = cbase + g * 64 + u * 16
          sv = srcc_v[pl.ds(base, 16)]
          dv = dstc_v[pl.ds(base, 16)]
          msk = plsc.bitcast(sv - lo, jnp.uint32) < jnp.uint32(npw)
          packed = sv | (dv << 14)
          mi = jnp.where(msk, jnp.int32(1), jnp.int32(0))
          csum = plsc.cumsum(mi)
          plsc.store_scatter(mpk_v, [cur + csum - mi], packed, mask=msk)
          cur = cur + plsc.all_reduce_population_count(msk)[0]
        return cur
      cursor = lax.fori_loop(0, K // 64, scan_g, cursor)

      nfull = cursor // G

      @pl.when(nfull > 0)
      def _():
        start_gather(0, gidx_a, drows_a, sem_a)

      def flush(m, _):
        gpar = lax.rem(m, 2)

        @pl.when(m + 1 < nfull)
        def _():
          @pl.when(gpar == 0)
          def _():
            start_gather((m + 1) * G, gidx_b, drows_b, sem_b)
          @pl.when(gpar == 1)
          def _():
            start_gather((m + 1) * G, gidx_a, drows_a, sem_a)

        @pl.when(gpar == 0)
        def _():
          accum_group(m * G, G, gidx_a, drows_a, sem_a, True)
        @pl.when(gpar == 1)
        def _():
          accum_group(m * G, G, gidx_b, drows_b, sem_b, True)
        return 0
      lax.fori_loop(0, nfull, flush, 0)

      rem = cursor - nfull * G

      def shift(j, _):
        mpk_v[pl.ds(j * 16, 16)] = mpk_v[pl.ds(nfull * G + j * 16, 16)]
        return 0
      lax.fori_loop(0, jnp.where(nfull > 0, (rem + 15) // 16, 0), shift, 0)
      return rem

    cursor = lax.fori_loop(0, nchunks, chunk_body, jnp.int32(0))

    @pl.when(cursor > 0)
    def _():
      start_gather(0, gidx_a, drows_a, sem_a)
      accum_group(0, cursor, gidx_a, drows_a, sem_a, False)

    pltpu.sync_copy(acc_v.at[pl.ds(0, npw * feat)],
                    out_hbm.at[pl.ds((h * n_pad + lo) * feat, npw * feat)])

  return segmax, n_pad


def _tc_fused(xf, sm_a, sm_b, w_even_t, w_odd_t, b2):
  n, feat = xf.shape
  blk = 2000
  assert n % blk == 0

  def body(xf_ref, sa_ref, sb_ref, we_ref, wo_ref, b_ref, o_ref):
    xb = xf_ref[...]
    sm = jnp.maximum(sa_ref[...], sb_ref[...])
    agg = jnp.where(sm == _NEG_INF, 0.0, sm - xb)
    y = jnp.dot(xb, we_ref[...], preferred_element_type=jnp.float32)
    y = y + jnp.dot(agg, wo_ref[...], preferred_element_type=jnp.float32)
    y = y + b_ref[...]
    o_ref[...] = jnp.maximum(y, 0.0)

  return pl.pallas_call(
      body,
      grid=(n // blk,),
      in_specs=[
          pl.BlockSpec((blk, feat), lambda i: (i, 0)),
          pl.BlockSpec((blk, feat), lambda i: (i, 0)),
          pl.BlockSpec((blk, feat), lambda i: (i, 0)),
          pl.BlockSpec((feat, feat), lambda i: (0, 0)),
          pl.BlockSpec((feat, feat), lambda i: (0, 0)),
          pl.BlockSpec((1, feat), lambda i: (0, 0)),
      ],
      out_specs=pl.BlockSpec((blk, feat), lambda i: (i, 0)),
      out_shape=jax.ShapeDtypeStruct((n, feat), jnp.float32),
  )(xf, sm_a, sm_b, w_even_t, w_odd_t, b2)


def kernel(x, edge_index, W, b):
  bsz, feat, n, _ = x.shape
  n_edges = edge_index.shape[1]
  assert bsz == 1

  xf = jnp.transpose(x[0, :, :, 0])
  src = edge_index[0].astype(jnp.int32)
  dst = edge_index[1].astype(jnp.int32)

  sc_segmax, n_pad = _make_sc_segmax(n, n_edges, feat)
  sm_flat = sc_segmax(xf, src, dst)
  sm2 = sm_flat.reshape(2, n_pad, feat)
  sm_a = sm2[0, :n]
  sm_b = sm2[1, :n]

  w_even_t = jnp.transpose(W[:, 0::2])
  w_odd_t = jnp.transpose(W[:, 1::2])
  y = _tc_fused(xf, sm_a, sm_b, w_even_t, w_odd_t, b.reshape(1, feat))
  return jnp.transpose(y)[None, :, :, None]


# speedup vs baseline: 1.1090x; 1.0433x over previous
import functools

import jax
import jax.numpy as jnp
from jax import lax
from jax.experimental import pallas as pl
from jax.experimental.pallas import tpu as pltpu
from jax.experimental.pallas import tpu_sc as plsc

_NEG_INF = float("-inf")


def _make_sc_segmax(n_nodes, n_edges, feat):
  info = plsc.get_sparse_core_info()
  nc, ns = info.num_cores, info.num_subcores
  nw = nc * ns
  nr = nw // 2
  npw = -(-n_nodes // nr)
  n_pad = npw * nr
  half = n_edges // 2
  K = 3200
  assert half % K == 0 and K % 128 == 0
  G = 128
  M = K + G + 16
  assert M % 16 == 0
  vpr = feat // 16

  mesh = plsc.VectorSubcoreMesh(core_axis_name="c", subcore_axis_name="s")

  @functools.partial(
      pl.kernel,
      mesh=mesh,
      compiler_params=pltpu.CompilerParams(needs_layout_passes=False),
      out_type=jax.ShapeDtypeStruct((2 * n_pad * feat,), jnp.float32),
      scratch_types=[
          pltpu.VMEM(((npw + 1) * feat,), jnp.float32),
          pltpu.VMEM((2 * K,), jnp.int32),
          pltpu.VMEM((2 * K,), jnp.int32),
          pltpu.VMEM((M,), jnp.int32),
          pltpu.VMEM((G,), jnp.int32),
          pltpu.VMEM((G,), jnp.int32),
          pltpu.VMEM((G, feat), jnp.float32),
          pltpu.VMEM((G, feat), jnp.float32),
          pltpu.SemaphoreType.DMA,
          pltpu.SemaphoreType.DMA,
          pltpu.SemaphoreType.DMA,
          pltpu.SemaphoreType.DMA,
      ],
  )
  def segmax(xf_hbm, src_hbm, dst_hbm, out_hbm,
             acc_v, srcc_v, dstc_v, mpk_v, gidx_a, gidx_b, drows_a, drows_b,
             sem_s, sem_d, sem_a, sem_b):
    h = lax.axis_index("c")
    rng = lax.axis_index("s")
    lo = rng * npw
    ebase = h * half
    nchunks = half // K

    def init_acc(i, _):
      acc_v[pl.ds(i * 16, 16)] = jnp.full((16,), _NEG_INF, jnp.float32)
      return 0
    lax.fori_loop(0, (npw + 1) * feat // 16, init_acc, 0)

    zeros16 = jnp.zeros((16,), jnp.int32)

    def init_m(i, _):
      mpk_v[pl.ds(i * 16, 16)] = zeros16
      return 0
    lax.fori_loop(0, M // 16, init_m, 0)

    lane_iota = lax.iota(jnp.int32, 16)

    def chunk_copy(t, par, sem, hbm, buf):
      return pltpu.make_async_copy(
          hbm.at[pl.ds(ebase + t * K, K)], buf.at[pl.ds(par * K, K)], sem)

    def start_gather(pos, gidx, drows, sem):
      def unpack(j, _):
        gidx[pl.ds(j * 16, 16)] = mpk_v[pl.ds(pos + j * 16, 16)] >> 14
        return 0
      lax.fori_loop(0, G // 16, unpack, 0)
      pltpu.make_async_copy(xf_hbm.at[gidx], drows, sem).start()

    def accum_group(pos, nvalid, gidx, drows, sem, full):
      pltpu.make_async_copy(xf_hbm.at[gidx], drows, sem).wait()

      def per_16(g, _):
        seg_v = mpk_v[pl.ds(pos + g * 16, 16)] & 0x3FFF
        if full:
          off_v = seg_v - lo
        else:
          m_v = g * 16 + lane_iota
          off_v = jnp.where(m_v < nvalid, seg_v - lo, npw)
        base_v = off_v * feat
        for lane in range(16):
          base = base_v[lane]
          m = g * 16 + lane
          for k in range(vpr):
            sl = pl.ds(base + k * 16, 16)
            acc_v[sl] = jnp.maximum(acc_v[sl], drows[m, pl.ds(k * 16, 16)])
        return 0
      lax.fori_loop(0, G // 16, per_16, 0)

    chunk_copy(0, 0, sem_s, src_hbm, srcc_v).start()
    chunk_copy(0, 0, sem_d, dst_hbm, dstc_v).start()

    def chunk_body(t, cursor):
      par = lax.rem(t, 2)
      chunk_copy(t, par, sem_s, src_hbm, srcc_v).wait()
      chunk_copy(t, par, sem_d, dst_hbm, dstc_v).wait()

      @pl.when(t + 1 < nchunks)
      def _():
        chunk_copy(t + 1, 1 - par, sem_s, src_hbm, srcc_v).start()
        chunk_copy(t + 1, 1 - par, sem_d, dst_hbm, dstc_v).start()

      cbase = par * K

      def scan_g(g, cur):
        for u in range(8):
          base = cbase + g * 128 + u * 16
          sv = srcc_v[pl.ds(base, 16)]
          dv = dstc_v[pl.ds(base, 16)]
          msk = plsc.bitcast(sv - lo, jnp.uint32) < jnp.uint32(npw)
          packed = sv | (dv << 14)
          mi = jnp.where(msk, jnp.int32(1), jnp.int32(0))
          csum = plsc.cumsum(mi)
          plsc.store_scatter(mpk_v, [cur + csum - mi], packed, mask=msk)
          cur = cur + plsc.all_reduce_population_count(msk)[0]
        return cur
      cursor = lax.fori_loop(0, K // 128, scan_g, cursor)

      nfull = cursor // G

      @pl.when(nfull > 0)
      def _():
        start_gather(0, gidx_a, drows_a, sem_a)

      def flush(m, _):
        gpar = lax.rem(m, 2)

        @pl.when(m + 1 < nfull)
        def _():
          @pl.when(gpar == 0)
          def _():
            start_gather((m + 1) * G, gidx_b, drows_b, sem_b)
          @pl.when(gpar == 1)
          def _():
            start_gather((m + 1) * G, gidx_a, drows_a, sem_a)

        @pl.when(gpar == 0)
        def _():
          accum_group(m * G, G, gidx_a, drows_a, sem_a, True)
        @pl.when(gpar == 1)
        def _():
          accum_group(m * G, G, gidx_b, drows_b, sem_b, True)
        return 0
      lax.fori_loop(0, nfull, flush, 0)

      rem = cursor - nfull * G

      def shift(j, _):
        mpk_v[pl.ds(j * 16, 16)] = mpk_v[pl.ds(nfull * G + j * 16, 16)]
        return 0
      lax.fori_loop(0, jnp.where(nfull > 0, (rem + 15) // 16, 0), shift, 0)
      return rem

    cursor = lax.fori_loop(0, nchunks, chunk_body, jnp.int32(0))

    @pl.when(cursor > 0)
    def _():
      start_gather(0, gidx_a, drows_a, sem_a)
      accum_group(0, cursor, gidx_a, drows_a, sem_a, False)

    pltpu.sync_copy(acc_v.at[pl.ds(0, npw * feat)],
                    out_hbm.at[pl.ds((h * n_pad + lo) * feat, npw * feat)])

  return segmax, n_pad


def _tc_fused(xf, sm_a, sm_b, w_even_t, w_odd_t, b2):
  n, feat = xf.shape
  blk = 2000
  assert n % blk == 0

  def body(xf_ref, sa_ref, sb_ref, we_ref, wo_ref, b_ref, o_ref):
    xb = xf_ref[...]
    sm = jnp.maximum(sa_ref[...], sb_ref[...])
    agg = jnp.where(sm == _NEG_INF, 0.0, sm - xb)
    y = jnp.dot(xb, we_ref[...], preferred_element_type=jnp.float32)
    y = y + jnp.dot(agg, wo_ref[...], preferred_element_type=jnp.float32)
    y = y + b_ref[...]
    o_ref[...] = jnp.maximum(y, 0.0)

  return pl.pallas_call(
      body,
      grid=(n // blk,),
      in_specs=[
          pl.BlockSpec((blk, feat), lambda i: (i, 0)),
          pl.BlockSpec((blk, feat), lambda i: (i, 0)),
          pl.BlockSpec((blk, feat), lambda i: (i, 0)),
          pl.BlockSpec((feat, feat), lambda i: (0, 0)),
          pl.BlockSpec((feat, feat), lambda i: (0, 0)),
          pl.BlockSpec((1, feat), lambda i: (0, 0)),
      ],
      out_specs=pl.BlockSpec((blk, feat), lambda i: (i, 0)),
      out_shape=jax.ShapeDtypeStruct((n, feat), jnp.float32),
  )(xf, sm_a, sm_b, w_even_t, w_odd_t, b2)


def kernel(x, edge_index, W, b):
  bsz, feat, n, _ = x.shape
  n_edges = edge_index.shape[1]
  assert bsz == 1

  xf = jnp.transpose(x[0, :, :, 0])
  src = edge_index[0].astype(jnp.int32)
  dst = edge_index[1].astype(jnp.int32)

  sc_segmax, n_pad = _make_sc_segmax(n, n_edges, feat)
  sm_flat = sc_segmax(xf, src, dst)
  sm2 = sm_flat.reshape(2, n_pad, feat)
  sm_a = sm2[0, :n]
  sm_b = sm2[1, :n]

  w_even_t = jnp.transpose(W[:, 0::2])
  w_odd_t = jnp.transpose(W[:, 1::2])
  y = _tc_fused(xf, sm_a, sm_b, w_even_t, w_odd_t, b.reshape(1, feat))
  return jnp.transpose(y)[None, :, :, None]


# speedup vs baseline: 1.1151x; 1.0055x over previous
import functools

import jax
import jax.numpy as jnp
from jax import lax
from jax.experimental import pallas as pl
from jax.experimental.pallas import tpu as pltpu
from jax.experimental.pallas import tpu_sc as plsc

_NEG_INF = float("-inf")


def _make_sc_segmax(n_nodes, n_edges, feat):
  info = plsc.get_sparse_core_info()
  nc, ns = info.num_cores, info.num_subcores
  nw = nc * ns
  nr = nw // 2
  npw = -(-n_nodes // nr)
  n_pad = npw * nr
  half = n_edges // 2
  K = 3200
  assert half % K == 0 and K % 128 == 0
  G = 128
  M = K + G + 16
  assert M % 16 == 0
  vpr = feat // 16

  mesh = plsc.VectorSubcoreMesh(core_axis_name="c", subcore_axis_name="s")

  @functools.partial(
      pl.kernel,
      mesh=mesh,
      compiler_params=pltpu.CompilerParams(needs_layout_passes=False),
      out_type=jax.ShapeDtypeStruct((2 * n_pad * feat,), jnp.float32),
      scratch_types=[
          pltpu.VMEM(((npw + 1) * feat,), jnp.float32),
          pltpu.VMEM((2 * K,), jnp.int32),
          pltpu.VMEM((2 * K,), jnp.int32),
          pltpu.VMEM((M,), jnp.int32),
          pltpu.VMEM((G,), jnp.int32),
          pltpu.VMEM((G,), jnp.int32),
          pltpu.VMEM((G, feat), jnp.float32),
          pltpu.VMEM((G, feat), jnp.float32),
          pltpu.SemaphoreType.DMA,
          pltpu.SemaphoreType.DMA,
          pltpu.SemaphoreType.DMA,
          pltpu.SemaphoreType.DMA,
      ],
  )
  def segmax(xf_hbm, src_hbm, dst_hbm, out_hbm,
             acc_v, srcc_v, dstc_v, mpk_v, gidx_a, gidx_b, drows_a, drows_b,
             sem_s, sem_d, sem_a, sem_b):
    h = lax.axis_index("c")
    rng = lax.axis_index("s")
    lo = rng * npw
    ebase = h * half
    nchunks = half // K

    def init_acc(i, _):
      acc_v[pl.ds(i * 16, 16)] = jnp.full((16,), _NEG_INF, jnp.float32)
      return 0
    lax.fori_loop(0, (npw + 1) * feat // 16, init_acc, 0)

    zeros16 = jnp.zeros((16,), jnp.int32)

    def init_m(i, _):
      mpk_v[pl.ds(i * 16, 16)] = zeros16
      return 0
    lax.fori_loop(0, M // 16, init_m, 0)

    lane_iota = lax.iota(jnp.int32, 16)

    def chunk_copy(t, par, sem, hbm, buf):
      return pltpu.make_async_copy(
          hbm.at[pl.ds(ebase + t * K, K)], buf.at[pl.ds(par * K, K)], sem)

    def start_gather(pos, gidx, drows, sem):
      def unpack(j, _):
        gidx[pl.ds(j * 16, 16)] = mpk_v[pl.ds(pos + j * 16, 16)] >> 14
        return 0
      lax.fori_loop(0, G // 16, unpack, 0)
      pltpu.make_async_copy(xf_hbm.at[gidx], drows, sem).start()

    def accum_group(pos, nvalid, gidx, drows, sem, full):
      pltpu.make_async_copy(xf_hbm.at[gidx], drows, sem).wait()

      def per_16(g, _):
        seg_v = mpk_v[pl.ds(pos + g * 16, 16)] & 0x3FFF
        if full:
          off_v = seg_v - lo
        else:
          m_v = g * 16 + lane_iota
          off_v = jnp.where(m_v < nvalid, seg_v - lo, npw)
        base_v = off_v * feat
        for lane in range(16):
          base = base_v[lane]
          m = g * 16 + lane
          for k in range(vpr):
            sl = pl.ds(base + k * 16, 16)
            acc_v[sl] = jnp.maximum(acc_v[sl], drows[m, pl.ds(k * 16, 16)])
        return 0
      lax.fori_loop(0, G // 16, per_16, 0)

    chunk_copy(0, 0, sem_s, src_hbm, srcc_v).start()
    chunk_copy(0, 0, sem_d, dst_hbm, dstc_v).start()

    def chunk_body(t, cursor):
      par = lax.rem(t, 2)
      chunk_copy(t, par, sem_s, src_hbm, srcc_v).wait()
      chunk_copy(t, par, sem_d, dst_hbm, dstc_v).wait()

      @pl.when(t + 1 < nchunks)
      def _():
        chunk_copy(t + 1, 1 - par, sem_s, src_hbm, srcc_v).start()
        chunk_copy(t + 1, 1 - par, sem_d, dst_hbm, dstc_v).start()

      cbase = par * K

      def scan_g(g, cur_v):
        for u in range(8):
          base = cbase + g * 128 + u * 16
          sv = srcc_v[pl.ds(base, 16)]
          dv = dstc_v[pl.ds(base, 16)]
          msk = plsc.bitcast(sv - lo, jnp.uint32) < jnp.uint32(npw)
          packed = sv | (dv << 14)
          mi = jnp.where(msk, jnp.int32(1), jnp.int32(0))
          csum = plsc.cumsum(mi)
          plsc.store_scatter(mpk_v, [cur_v + csum - mi], packed, mask=msk)
          cur_v = cur_v + plsc.all_reduce_population_count(msk)
        return cur_v
      cursor_v = lax.fori_loop(
          0, K // 128, scan_g, jnp.zeros((16,), jnp.int32) + cursor)
      cursor = cursor_v[0]

      nfull = cursor // G

      @pl.when(nfull > 0)
      def _():
        start_gather(0, gidx_a, drows_a, sem_a)

      def flush(m, _):
        gpar = lax.rem(m, 2)

        @pl.when(m + 1 < nfull)
        def _():
          @pl.when(gpar == 0)
          def _():
            start_gather((m + 1) * G, gidx_b, drows_b, sem_b)
          @pl.when(gpar == 1)
          def _():
            start_gather((m + 1) * G, gidx_a, drows_a, sem_a)

        @pl.when(gpar == 0)
        def _():
          accum_group(m * G, G, gidx_a, drows_a, sem_a, True)
        @pl.when(gpar == 1)
        def _():
          accum_group(m * G, G, gidx_b, drows_b, sem_b, True)
        return 0
      lax.fori_loop(0, nfull, flush, 0)

      rem = cursor - nfull * G

      def shift(j, _):
        mpk_v[pl.ds(j * 16, 16)] = mpk_v[pl.ds(nfull * G + j * 16, 16)]
        return 0
      lax.fori_loop(0, jnp.where(nfull > 0, (rem + 15) // 16, 0), shift, 0)
      return rem

    cursor = lax.fori_loop(0, nchunks, chunk_body, jnp.int32(0))

    @pl.when(cursor > 0)
    def _():
      start_gather(0, gidx_a, drows_a, sem_a)
      accum_group(0, cursor, gidx_a, drows_a, sem_a, False)

    pltpu.sync_copy(acc_v.at[pl.ds(0, npw * feat)],
                    out_hbm.at[pl.ds((h * n_pad + lo) * feat, npw * feat)])

  return segmax, n_pad


def _tc_fused(xf, sm_a, sm_b, w_even_t, w_odd_t, b2):
  n, feat = xf.shape
  blk = 2000
  assert n % blk == 0

  def body(xf_ref, sa_ref, sb_ref, we_ref, wo_ref, b_ref, o_ref):
    xb = xf_ref[...]
    sm = jnp.maximum(sa_ref[...], sb_ref[...])
    agg = jnp.where(sm == _NEG_INF, 0.0, sm - xb)
    y = jnp.dot(xb, we_ref[...], preferred_element_type=jnp.float32)
    y = y + jnp.dot(agg, wo_ref[...], preferred_element_type=jnp.float32)
    y = y + b_ref[...]
    o_ref[...] = jnp.maximum(y, 0.0)

  return pl.pallas_call(
      body,
      grid=(n // blk,),
      in_specs=[
          pl.BlockSpec((blk, feat), lambda i: (i, 0)),
          pl.BlockSpec((blk, feat), lambda i: (i, 0)),
          pl.BlockSpec((blk, feat), lambda i: (i, 0)),
          pl.BlockSpec((feat, feat), lambda i: (0, 0)),
          pl.BlockSpec((feat, feat), lambda i: (0, 0)),
          pl.BlockSpec((1, feat), lambda i: (0, 0)),
      ],
      out_specs=pl.BlockSpec((blk, feat), lambda i: (i, 0)),
      out_shape=jax.ShapeDtypeStruct((n, feat), jnp.float32),
  )(xf, sm_a, sm_b, w_even_t, w_odd_t, b2)


def kernel(x, edge_index, W, b):
  bsz, feat, n, _ = x.shape
  n_edges = edge_index.shape[1]
  assert bsz == 1

  xf = jnp.transpose(x[0, :, :, 0])
  src = edge_index[0].astype(jnp.int32)
  dst = edge_index[1].astype(jnp.int32)

  sc_segmax, n_pad = _make_sc_segmax(n, n_edges, feat)
  sm_flat = sc_segmax(xf, src, dst)
  sm2 = sm_flat.reshape(2, n_pad, feat)
  sm_a = sm2[0, :n]
  sm_b = sm2[1, :n]

  w_even_t = jnp.transpose(W[:, 0::2])
  w_odd_t = jnp.transpose(W[:, 1::2])
  y = _tc_fused(xf, sm_a, sm_b, w_even_t, w_odd_t, b.reshape(1, feat))
  return jnp.transpose(y)[None, :, :, None]
